# baseline (speedup 1.0000x reference)
import functools

import jax
import jax.numpy as jnp
from jax import lax
from jax.experimental import pallas as pl
from jax.experimental.pallas import tpu as pltpu
from jax.experimental.pallas import tpu_sc as plsc

N = 10000
IN = 128
H = 256
HH = H // 2
OUT = 128
G = 64
NEG = 0.2
BM = 512
NP = ((N + BM - 1) // BM) * BM
NT = 16
RPT = NP // NT
C = 96
ZR = 128
F32 = jnp.float32
I32 = jnp.int32
_PREC = lax.Precision.DEFAULT


def _dot(a, b, ca, cb):
    return lax.dot_general(a, b, ((ca, cb), ((), ())),
                           precision=_PREC, preferred_element_type=F32)


def _gat_tail(h, wc_ref, asv_ref, adv_ref, g0_ref, g1_ref, ar_ref, br_ref):
    g = _dot(h, wc_ref[...], (1,), (0,))
    g0_ref[...] = g[:, :HH]
    g1_ref[...] = g[:, HH:]
    ar_ref[...] = _dot(asv_ref[...], g, (1,), (1,))
    br_ref[...] = _dot(adv_ref[...], g, (1,), (1,))


def _front_body(x_ref, w0_ref, b0_ref, wc_ref, asv_ref, adv_ref,
                g0_ref, g1_ref, ar_ref, br_ref):
    h = jnp.maximum(_dot(x_ref[...], w0_ref[...], (1,), (0,)) + b0_ref[...], 0.0)
    _gat_tail(h, wc_ref, asv_ref, adv_ref, g0_ref, g1_ref, ar_ref, br_ref)


def _mid_body(a0_ref, a1_ref, dn_ref, bc_ref, wc_ref, asv_ref, adv_ref,
              g0_ref, g1_ref, ar_ref, br_ref):
    acc = jnp.concatenate([a0_ref[...], a1_ref[...]], axis=1)
    dn = (dn_ref[0, :] + dn_ref[1, :] + 1e-16)[:, None]
    h = jnp.maximum(acc / dn + bc_ref[...], 0.0)
    _gat_tail(h, wc_ref, asv_ref, adv_ref, g0_ref, g1_ref, ar_ref, br_ref)


def _pool_body(a0_ref, a1_ref, dn_ref, bt_ref, bc_ref, w1_ref, b1_ref,
               out_ref, sums_ref):
    i = pl.program_id(0)
    nsteps = pl.num_programs(0)

    @pl.when(i == 0)
    def _():
        sums_ref[...] = jnp.zeros_like(sums_ref)

    acc = jnp.concatenate([a0_ref[...], a1_ref[...]], axis=1)
    dn = (dn_ref[0, :] + dn_ref[1, :] + 1e-16)[:, None]
    h = jnp.maximum(acc / dn + bc_ref[...], 0.0)
    seg = bt_ref[0, pl.ds(i * BM, BM)]
    oh = (seg[:, None] == lax.broadcasted_iota(I32, (BM, G), 1)).astype(F32)
    sums_ref[...] += _dot(oh, h, (0,), (0,))

    @pl.when(i == nsteps - 1)
    def _():
        ball = bt_ref[0, :]
        cnt = jnp.sum(
            (ball[None, :] == lax.broadcasted_iota(I32, (G, NP), 0)).astype(F32),
            axis=1)
        pooled = sums_ref[...] / jnp.clip(cnt, 1.0, None)[:, None]
        out_ref[...] = _dot(pooled, w1_ref[...], (1,), (0,)) + b1_ref[...]


def _tc_front(xp, W0, b0, Wc, asv, adv):
    grid = (NP // BM,)
    return pl.pallas_call(
        _front_body,
        grid=grid,
        in_specs=[
            pl.BlockSpec((BM, IN), lambda i: (i, 0)),
            pl.BlockSpec((IN, H), lambda i: (0, 0)),
            pl.BlockSpec((1, H), lambda i: (0, 0)),
            pl.BlockSpec((H, H), lambda i: (0, 0)),
            pl.BlockSpec((1, H), lambda i: (0, 0)),
            pl.BlockSpec((1, H), lambda i: (0, 0)),
        ],
        out_specs=[
            pl.BlockSpec((BM, HH), lambda i: (i, 0)),
            pl.BlockSpec((BM, HH), lambda i: (i, 0)),
            pl.BlockSpec((1, BM), lambda i: (0, i)),
            pl.BlockSpec((1, BM), lambda i: (0, i)),
        ],
        out_shape=[
            jax.ShapeDtypeStruct((NP, HH), F32),
            jax.ShapeDtypeStruct((NP, HH), F32),
            jax.ShapeDtypeStruct((1, NP), F32),
            jax.ShapeDtypeStruct((1, NP), F32),
        ],
    )(xp, W0, b0.reshape(1, H), Wc, asv.reshape(1, H), adv.reshape(1, H))


def _tc_mid(a0, a1, dn, bc, Wc, asv, adv):
    grid = (NP // BM,)
    return pl.pallas_call(
        _mid_body,
        grid=grid,
        in_specs=[
            pl.BlockSpec((BM, HH), lambda i: (i, 0)),
            pl.BlockSpec((BM, HH), lambda i: (i, 0)),
            pl.BlockSpec((2, BM), lambda i: (0, i)),
            pl.BlockSpec((1, H), lambda i: (0, 0)),
            pl.BlockSpec((H, H), lambda i: (0, 0)),
            pl.BlockSpec((1, H), lambda i: (0, 0)),
            pl.BlockSpec((1, H), lambda i: (0, 0)),
        ],
        out_specs=[
            pl.BlockSpec((BM, HH), lambda i: (i, 0)),
            pl.BlockSpec((BM, HH), lambda i: (i, 0)),
            pl.BlockSpec((1, BM), lambda i: (0, i)),
            pl.BlockSpec((1, BM), lambda i: (0, i)),
        ],
        out_shape=[
            jax.ShapeDtypeStruct((NP, HH), F32),
            jax.ShapeDtypeStruct((NP, HH), F32),
            jax.ShapeDtypeStruct((1, NP), F32),
            jax.ShapeDtypeStruct((1, NP), F32),
        ],
    )(a0, a1, dn, bc.reshape(1, H), Wc,
      asv.reshape(1, H), adv.reshape(1, H))


def _tc_pool(a0, a1, dn, bt, bc, W1, b1):
    grid = (NP // BM,)
    return pl.pallas_call(
        _pool_body,
        grid=grid,
        in_specs=[
            pl.BlockSpec((BM, HH), lambda i: (i, 0)),
            pl.BlockSpec((BM, HH), lambda i: (i, 0)),
            pl.BlockSpec((2, BM), lambda i: (0, i)),
            pl.BlockSpec((1, NP), lambda i: (0, 0)),
            pl.BlockSpec((1, H), lambda i: (0, 0)),
            pl.BlockSpec((H, OUT), lambda i: (0, 0)),
            pl.BlockSpec((1, OUT), lambda i: (0, 0)),
        ],
        out_specs=pl.BlockSpec((G, OUT), lambda i: (0, 0)),
        out_shape=jax.ShapeDtypeStruct((G, OUT), F32),
        scratch_shapes=[pltpu.VMEM((G, H), F32)],
    )(a0, a1, dn, bt.reshape(1, NP), bc.reshape(1, H),
      W1, b1.reshape(1, OUT))


def _edge_pass(sd, asrc, adst, g0, g1):
    nch_tot = sd.shape[0]
    nch = nch_tot // NT
    ntrip = nch // 3
    mesh = plsc.VectorSubcoreMesh(core_axis_name="c", subcore_axis_name="s")

    @functools.partial(
        pl.kernel,
        mesh=mesh,
        out_type=(
            jax.ShapeDtypeStruct((2 * NP, HH), F32),
            jax.ShapeDtypeStruct((2, NP), F32),
        ),
        scratch_types=[
            pltpu.VMEM((2, C), I32), pltpu.VMEM((2, C), I32),
            pltpu.VMEM((2, C), I32),
            pltpu.VMEM((C,), F32), pltpu.VMEM((C,), F32),
            pltpu.VMEM((C,), F32),
            pltpu.VMEM((C,), F32), pltpu.VMEM((C,), F32),
            pltpu.VMEM((C,), F32),
            pltpu.VMEM((C,), F32), pltpu.VMEM((C,), F32),
            pltpu.VMEM((C,), F32),
            pltpu.VMEM((C, HH), F32), pltpu.VMEM((C, HH), F32),
            pltpu.VMEM((C, HH), F32),
            pltpu.VMEM((RPT,), F32),
            pltpu.VMEM_SHARED((NP, HH), F32),
            pltpu.VMEM_SHARED((NP,), F32),
            pltpu.SemaphoreType.DMA, pltpu.SemaphoreType.DMA,
            pltpu.SemaphoreType.DMA,
            pltpu.SemaphoreType.DMA, pltpu.SemaphoreType.DMA,
            pltpu.SemaphoreType.DMA,
        ],
    )
    def k(sd_h, asrc_h, adst_h, g0_h, g1_h, acc_out, den_out,
          sd0, sd1, sd2, av0, av1, av2, bv0, bv1, bv2, ex0, ex1, ex2,
          rw0, rw1, rw2, zvec, acc_sp, den_sp,
          semg0, semg1, semg2, semr0, semr1, semr2):
        c = lax.axis_index("c")
        s = lax.axis_index("s")
        sd_ = (sd0, sd1, sd2)
        av = (av0, av1, av2)
        bv = (bv0, bv1, bv2)
        ex = (ex0, ex1, ex2)
        rw = (rw0, rw1, rw2)
        semg = (semg0, semg1, semg2)
        semr = (semr0, semr1, semr2)
        z16 = jnp.zeros((16,), F32)

        @plsc.parallel_loop(0, C)
        def _(i):
            for j in range(HH // 16):
                rw0[i, pl.ds(j * 16, 16)] = z16

        @plsc.parallel_loop(0, RPT // 16)
        def _(i):
            zvec[pl.ds(i * 16, 16)] = z16

        r0 = s * RPT
        nfull = RPT // C
        for b in range(nfull):
            pltpu.sync_copy(rw0, acc_sp.at[pl.ds(r0 + b * C, C), :])
        rem = RPT - nfull * C
        if rem:
            pltpu.sync_copy(rw0.at[pl.ds(0, rem), :],
                            acc_sp.at[pl.ds(r0 + nfull * C, rem), :])
        pltpu.sync_copy(zvec, den_sp.at[pl.ds(r0, RPT)])
        plsc.subcore_barrier()

        kbase = s * nch

        def start_rows(b):
            @pl.when(c == 0)
            def _():
                pltpu.async_copy(g0_h.at[sd_[b].at[0]], rw[b], semr[b])

            @pl.when(c == 1)
            def _():
                pltpu.async_copy(g1_h.at[sd_[b].at[0]], rw[b], semr[b])

        def wait_rows(b):
            @pl.when(c == 0)
            def _():
                pltpu.make_async_copy(g0_h.at[sd_[b].at[0]], rw[b],
                                      semr[b]).wait()

            @pl.when(c == 1)
            def _():
                pltpu.make_async_copy(g1_h.at[sd_[b].at[0]], rw[b],
                                      semr[b]).wait()

        def prep(ci, b):
            pltpu.sync_copy(sd_h.at[kbase + ci], sd_[b])
            start_rows(b)
            pltpu.async_copy(asrc_h.at[sd_[b].at[0]], av[b], semg[b])
            pltpu.async_copy(adst_h.at[sd_[b].at[1]], bv[b], semg[b])

        def drain(ci, b):
            rows = rw[b]
            exb = ex[b]
            pltpu.make_async_copy(asrc_h.at[sd_[b].at[0]], av[b],
                                  semg[b]).wait()
            pltpu.make_async_copy(adst_h.at[sd_[b].at[1]], bv[b],
                                  semg[b]).wait()

            @plsc.parallel_loop(0, C // 16)
            def _(j):
                sl = pl.ds(j * 16, 16)
                sv = av[b][sl] + bv[b][sl]
                ev = jnp.where(sv >= 0.0, sv, sv * NEG)
                exb[sl] = jnp.exp(ev)

            @pl.when(c == lax.rem(ci, 2))
            def _():
                pltpu.sync_copy(exb, den_sp.at[sd_[b].at[1]], add=True)

            wait_rows(b)

            @plsc.parallel_loop(0, C // 16)
            def _(jb):
                exv = exb[pl.ds(jb * 16, 16)]
                for l in range(16):
                    sc16 = jnp.full((16,), exv[l], dtype=F32)
                    r = jb * 16 + l
                    for q in range(HH // 16):
                        sl = pl.ds(q * 16, 16)
                        rows[r, sl] = rows[r, sl] * sc16

            pltpu.sync_copy(rows, acc_sp.at[sd_[b].at[1]], add=True)

        prep(0, 0)
        prep(1, 1)

        def trip(t, _):
            base = 3 * t
            prep(base + 2, 2)
            drain(base, 0)

            @pl.when(t < ntrip - 1)
            def _():
                prep(base + 3, 0)

            drain(base + 1, 1)

            @pl.when(t < ntrip - 1)
            def _():
                prep(base + 4, 1)

            drain(base + 2, 2)
            return 0

        lax.fori_loop(0, ntrip, trip, 0)
        plsc.subcore_barrier()

        coff = c * NP
        pltpu.sync_copy(acc_sp.at[pl.ds(r0, RPT), :],
                        acc_out.at[pl.ds(coff + r0, RPT), :])
        pltpu.sync_copy(den_sp.at[pl.ds(r0, RPT)],
                        den_out.at[c, pl.ds(r0, RPT)])

    return k(sd, asrc, adst, g0, g1)


def kernel(x, edge_index, edge_attr, batch, W0, b0, Wc0, as0, ad0, bc0,
           Wc1, as1, ad1, bc1, W1, b1):
    del edge_attr
    E = edge_index.shape[1]
    etot = E + N
    EP = ((etot + NT * C - 1) // (NT * C)) * (NT * C)
    pad = EP - etot

    loop = jnp.arange(N, dtype=I32)
    src = jnp.concatenate([edge_index[0].astype(I32), loop,
                           jnp.zeros((pad,), I32)])
    dst = jnp.concatenate([edge_index[1].astype(I32), loop,
                           jnp.full((pad,), N, I32)])
    sd = jnp.stack([src.reshape(EP // C, C), dst.reshape(EP // C, C)], axis=1)
    xp = jnp.zeros((NP, IN), F32).at[:N].set(x)
    bt = jnp.concatenate([batch.astype(I32), jnp.full((NP - N,), G, I32)])

    g0, g1, ar, br = _tc_front(xp, W0, b0, Wc0, as0, ad0)
    acc, den = _edge_pass(sd, ar.reshape(NP), br.reshape(NP), g0, g1)

    g0, g1, ar, br = _tc_mid(acc[:NP], acc[NP:], den, bc0, Wc1, as1, ad1)
    acc, den = _edge_pass(sd, ar.reshape(NP), br.reshape(NP), g0, g1)

    return _tc_pool(acc[:NP], acc[NP:], den, bt, bc1, W1, b1)


# speedup vs baseline: 1.0454x; 1.0454x over previous
import functools

import jax
import jax.numpy as jnp
from jax import lax
from jax.experimental import pallas as pl
from jax.experimental.pallas import tpu as pltpu
from jax.experimental.pallas import tpu_sc as plsc

N = 10000
IN = 128
H = 256
HH = H // 2
OUT = 128
G = 64
NEG = 0.2
BM = 512
NP = ((N + BM - 1) // BM) * BM
NT = 16
RPT = NP // NT
C = 128
ZR = 128
F32 = jnp.float32
I32 = jnp.int32
_PREC = lax.Precision.DEFAULT


def _dot(a, b, ca, cb):
    return lax.dot_general(a, b, ((ca, cb), ((), ())),
                           precision=_PREC, preferred_element_type=F32)


def _gat_tail(h, wc_ref, asv_ref, adv_ref, g0_ref, g1_ref, ar_ref, br_ref):
    g = _dot(h, wc_ref[...], (1,), (0,))
    g0_ref[...] = g[:, :HH]
    g1_ref[...] = g[:, HH:]
    ar_ref[...] = _dot(asv_ref[...], g, (1,), (1,))
    br_ref[...] = _dot(adv_ref[...], g, (1,), (1,))


def _front_body(x_ref, w0_ref, b0_ref, wc_ref, asv_ref, adv_ref,
                g0_ref, g1_ref, ar_ref, br_ref):
    h = jnp.maximum(_dot(x_ref[...], w0_ref[...], (1,), (0,)) + b0_ref[...], 0.0)
    _gat_tail(h, wc_ref, asv_ref, adv_ref, g0_ref, g1_ref, ar_ref, br_ref)


def _mid_body(a0_ref, a1_ref, dn_ref, bc_ref, wc_ref, asv_ref, adv_ref,
              g0_ref, g1_ref, ar_ref, br_ref):
    acc = jnp.concatenate([a0_ref[...], a1_ref[...]], axis=1)
    dn = (dn_ref[0, :] + dn_ref[1, :] + 1e-16)[:, None]
    h = jnp.maximum(acc / dn + bc_ref[...], 0.0)
    _gat_tail(h, wc_ref, asv_ref, adv_ref, g0_ref, g1_ref, ar_ref, br_ref)


def _pool_body(a0_ref, a1_ref, dn_ref, bt_ref, bc_ref, w1_ref, b1_ref,
               out_ref, sums_ref):
    i = pl.program_id(0)
    nsteps = pl.num_programs(0)

    @pl.when(i == 0)
    def _():
        sums_ref[...] = jnp.zeros_like(sums_ref)

    acc = jnp.concatenate([a0_ref[...], a1_ref[...]], axis=1)
    dn = (dn_ref[0, :] + dn_ref[1, :] + 1e-16)[:, None]
    h = jnp.maximum(acc / dn + bc_ref[...], 0.0)
    seg = bt_ref[0, pl.ds(i * BM, BM)]
    oh = (seg[:, None] == lax.broadcasted_iota(I32, (BM, G), 1)).astype(F32)
    sums_ref[...] += _dot(oh, h, (0,), (0,))

    @pl.when(i == nsteps - 1)
    def _():
        ball = bt_ref[0, :]
        cnt = jnp.sum(
            (ball[None, :] == lax.broadcasted_iota(I32, (G, NP), 0)).astype(F32),
            axis=1)
        pooled = sums_ref[...] / jnp.clip(cnt, 1.0, None)[:, None]
        out_ref[...] = _dot(pooled, w1_ref[...], (1,), (0,)) + b1_ref[...]


def _tc_front(xp, W0, b0, Wc, asv, adv):
    grid = (NP // BM,)
    return pl.pallas_call(
        _front_body,
        grid=grid,
        in_specs=[
            pl.BlockSpec((BM, IN), lambda i: (i, 0)),
            pl.BlockSpec((IN, H), lambda i: (0, 0)),
            pl.BlockSpec((1, H), lambda i: (0, 0)),
            pl.BlockSpec((H, H), lambda i: (0, 0)),
            pl.BlockSpec((1, H), lambda i: (0, 0)),
            pl.BlockSpec((1, H), lambda i: (0, 0)),
        ],
        out_specs=[
            pl.BlockSpec((BM, HH), lambda i: (i, 0)),
            pl.BlockSpec((BM, HH), lambda i: (i, 0)),
            pl.BlockSpec((1, BM), lambda i: (0, i)),
            pl.BlockSpec((1, BM), lambda i: (0, i)),
        ],
        out_shape=[
            jax.ShapeDtypeStruct((NP, HH), F32),
            jax.ShapeDtypeStruct((NP, HH), F32),
            jax.ShapeDtypeStruct((1, NP), F32),
            jax.ShapeDtypeStruct((1, NP), F32),
        ],
    )(xp, W0, b0.reshape(1, H), Wc, asv.reshape(1, H), adv.reshape(1, H))


def _tc_mid(a0, a1, dn, bc, Wc, asv, adv):
    grid = (NP // BM,)
    return pl.pallas_call(
        _mid_body,
        grid=grid,
        in_specs=[
            pl.BlockSpec((BM, HH), lambda i: (i, 0)),
            pl.BlockSpec((BM, HH), lambda i: (i, 0)),
            pl.BlockSpec((2, BM), lambda i: (0, i)),
            pl.BlockSpec((1, H), lambda i: (0, 0)),
            pl.BlockSpec((H, H), lambda i: (0, 0)),
            pl.BlockSpec((1, H), lambda i: (0, 0)),
            pl.BlockSpec((1, H), lambda i: (0, 0)),
        ],
        out_specs=[
            pl.BlockSpec((BM, HH), lambda i: (i, 0)),
            pl.BlockSpec((BM, HH), lambda i: (i, 0)),
            pl.BlockSpec((1, BM), lambda i: (0, i)),
            pl.BlockSpec((1, BM), lambda i: (0, i)),
        ],
        out_shape=[
            jax.ShapeDtypeStruct((NP, HH), F32),
            jax.ShapeDtypeStruct((NP, HH), F32),
            jax.ShapeDtypeStruct((1, NP), F32),
            jax.ShapeDtypeStruct((1, NP), F32),
        ],
    )(a0, a1, dn, bc.reshape(1, H), Wc,
      asv.reshape(1, H), adv.reshape(1, H))


def _tc_pool(a0, a1, dn, bt, bc, W1, b1):
    grid = (NP // BM,)
    return pl.pallas_call(
        _pool_body,
        grid=grid,
        in_specs=[
            pl.BlockSpec((BM, HH), lambda i: (i, 0)),
            pl.BlockSpec((BM, HH), lambda i: (i, 0)),
            pl.BlockSpec((2, BM), lambda i: (0, i)),
            pl.BlockSpec((1, NP), lambda i: (0, 0)),
            pl.BlockSpec((1, H), lambda i: (0, 0)),
            pl.BlockSpec((H, OUT), lambda i: (0, 0)),
            pl.BlockSpec((1, OUT), lambda i: (0, 0)),
        ],
        out_specs=pl.BlockSpec((G, OUT), lambda i: (0, 0)),
        out_shape=jax.ShapeDtypeStruct((G, OUT), F32),
        scratch_shapes=[pltpu.VMEM((G, H), F32)],
    )(a0, a1, dn, bt.reshape(1, NP), bc.reshape(1, H),
      W1, b1.reshape(1, OUT))


def _edge_pass(sd, asrc, adst, g0, g1):
    nch_tot = sd.shape[0]
    nch = nch_tot // NT
    npairs = nch // 2
    mesh = plsc.VectorSubcoreMesh(core_axis_name="c", subcore_axis_name="s")

    @functools.partial(
        pl.kernel,
        mesh=mesh,
        out_type=(
            jax.ShapeDtypeStruct((2 * NP, HH), F32),
            jax.ShapeDtypeStruct((2, NP), F32),
        ),
        scratch_types=[
            pltpu.VMEM((2, C), I32), pltpu.VMEM((2, C), I32),
            pltpu.VMEM((C,), F32), pltpu.VMEM((C,), F32),
            pltpu.VMEM((C,), F32), pltpu.VMEM((C,), F32),
            pltpu.VMEM((C,), F32), pltpu.VMEM((C,), F32),
            pltpu.VMEM((C, HH), F32), pltpu.VMEM((C, HH), F32),
            pltpu.VMEM((RPT,), F32),
            pltpu.VMEM_SHARED((NP, HH), F32),
            pltpu.VMEM_SHARED((NP,), F32),
            pltpu.SemaphoreType.DMA,
            pltpu.SemaphoreType.DMA,
            pltpu.SemaphoreType.DMA,
            pltpu.SemaphoreType.DMA,
        ],
    )
    def k(sd_h, asrc_h, adst_h, g0_h, g1_h, acc_out, den_out,
          sd0, sd1, av0, av1, bv0, bv1, ex0, ex1,
          rw0, rw1, zvec, acc_sp, den_sp, semg0, semg1, semr0, semr1):
        c = lax.axis_index("c")
        s = lax.axis_index("s")
        sd_ = (sd0, sd1)
        av = (av0, av1)
        bv = (bv0, bv1)
        ex = (ex0, ex1)
        rw = (rw0, rw1)
        semr = (semr0, semr1)
        semg = (semg0, semg1)
        z16 = jnp.zeros((16,), F32)

        @plsc.parallel_loop(0, ZR)
        def _(i):
            for j in range(HH // 16):
                rw0[i, pl.ds(j * 16, 16)] = z16

        @plsc.parallel_loop(0, RPT // 16)
        def _(i):
            zvec[pl.ds(i * 16, 16)] = z16

        r0 = s * RPT
        for b in range(RPT // ZR):
            pltpu.sync_copy(rw0, acc_sp.at[pl.ds(r0 + b * ZR, ZR), :])
        pltpu.sync_copy(zvec, den_sp.at[pl.ds(r0, RPT)])
        plsc.subcore_barrier()

        kbase = s * nch

        def start_rows(b):
            @pl.when(c == 0)
            def _():
                pltpu.async_copy(g0_h.at[sd_[b].at[0]], rw[b], semr[b])

            @pl.when(c == 1)
            def _():
                pltpu.async_copy(g1_h.at[sd_[b].at[0]], rw[b], semr[b])

        def wait_rows(b):
            @pl.when(c == 0)
            def _():
                pltpu.make_async_copy(g0_h.at[sd_[b].at[0]], rw[b],
                                      semr[b]).wait()

            @pl.when(c == 1)
            def _():
                pltpu.make_async_copy(g1_h.at[sd_[b].at[0]], rw[b],
                                      semr[b]).wait()

        def prep(ci, b):
            pltpu.sync_copy(sd_h.at[kbase + ci], sd_[b])
            start_rows(b)
            pltpu.async_copy(asrc_h.at[sd_[b].at[0]], av[b], semg[b])
            pltpu.async_copy(adst_h.at[sd_[b].at[1]], bv[b], semg[b])

        def drain(b):
            rows = rw[b]
            exb = ex[b]
            pltpu.make_async_copy(asrc_h.at[sd_[b].at[0]], av[b],
                                  semg[b]).wait()
            pltpu.make_async_copy(adst_h.at[sd_[b].at[1]], bv[b],
                                  semg[b]).wait()

            @plsc.parallel_loop(0, C // 16)
            def _(j):
                sl = pl.ds(j * 16, 16)
                sv = av[b][sl] + bv[b][sl]
                ev = jnp.where(sv >= 0.0, sv, sv * NEG)
                exb[sl] = jnp.exp(ev)

            @pl.when(c == b)
            def _():
                pltpu.sync_copy(exb, den_sp.at[sd_[b].at[1]], add=True)

            @plsc.parallel_loop(0, C // 16)
            def _(jb):
                exv = exb[pl.ds(jb * 16, 16)]
                for l in range(16):
                    sc16 = jnp.full((16,), exv[l], dtype=F32)
                    r = jb * 16 + l
                    for q in range(HH // 16):
                        sl = pl.ds(q * 16, 16)
                        rows[r, sl] = rows[r, sl] * sc16

            pltpu.sync_copy(rows, acc_sp.at[sd_[b].at[1]], add=True)

        prep(0, 0)

        def pair(p, _):
            prep(2 * p + 1, 1)
            wait_rows(0)
            drain(0)

            @pl.when(p < npairs - 1)
            def _():
                prep(2 * p + 2, 0)

            wait_rows(1)
            drain(1)
            return 0

        lax.fori_loop(0, npairs, pair, 0)
        plsc.subcore_barrier()

        coff = c * NP
        pltpu.sync_copy(acc_sp.at[pl.ds(r0, RPT), :],
                        acc_out.at[pl.ds(coff + r0, RPT), :])

        pltpu.sync_copy(den_sp.at[pl.ds(r0, RPT)],
                        den_out.at[c, pl.ds(r0, RPT)])

    return k(sd, asrc, adst, g0, g1)


def kernel(x, edge_index, edge_attr, batch, W0, b0, Wc0, as0, ad0, bc0,
           Wc1, as1, ad1, bc1, W1, b1):
    del edge_attr
    E = edge_index.shape[1]
    etot = E + N
    EP = ((etot + NT * C - 1) // (NT * C)) * (NT * C)
    pad = EP - etot

    loop = jnp.arange(N, dtype=I32)
    src = jnp.concatenate([edge_index[0].astype(I32), loop,
                           jnp.zeros((pad,), I32)])
    dst = jnp.concatenate([edge_index[1].astype(I32), loop,
                           jnp.full((pad,), N, I32)])
    sd = jnp.stack([src.reshape(EP // C, C), dst.reshape(EP // C, C)], axis=1)
    xp = jnp.zeros((NP, IN), F32).at[:N].set(x)
    bt = jnp.concatenate([batch.astype(I32), jnp.full((NP - N,), G, I32)])

    g0, g1, ar, br = _tc_front(xp, W0, b0, Wc0, as0, ad0)
    acc, den = _edge_pass(sd, ar.reshape(NP), br.reshape(NP), g0, g1)

    g0, g1, ar, br = _tc_mid(acc[:NP], acc[NP:], den, bc0, Wc1, as1, ad1)
    acc, den = _edge_pass(sd, ar.reshape(NP), br.reshape(NP), g0, g1)

    return _tc_pool(acc[:NP], acc[NP:], den, bt, bc1, W1, b1)
